# baseline (speedup 1.0000x reference)
import functools

import jax
import jax.numpy as jnp
from jax import lax
from jax.experimental import pallas as pl
from jax.experimental.pallas import tpu as pltpu
from jax.experimental.pallas import tpu_sc as plsc

_VOCAB = 100000
_HIDDEN = 768
_MAX_POS = 2048
_BATCH = 4
_SEQ = 2048

_L = 16
_NV = _HIDDEN // _L
_NW = 32
_S_PER_W = _SEQ // _NW
_CH = 32
_NCH_S = _S_PER_W // _CH
_INV_H = 1.0 / _HIDDEN
_EPS = 1e-12


def _lane_sum(v):
    dnums = lax.GatherDimensionNumbers(
        offset_dims=(), collapsed_slice_dims=(0,), start_index_map=(0,))
    for sh in (8, 4, 2, 1):
        idx = lax.iota(jnp.int32, _L) ^ sh
        v = v + lax.gather(v, idx[:, None], dnums, (1,),
                           mode=lax.GatherScatterMode.PROMISE_IN_BOUNDS)
    return v


def _rsqrt_vec(v):
    i = lax.bitcast_convert_type(v, jnp.int32)
    i = jnp.full((_L,), 0x5F3759DF, jnp.int32) - lax.shift_right_logical(
        i, jnp.full((_L,), 1, jnp.int32))
    y = lax.bitcast_convert_type(i, jnp.float32)
    half_v = v * 0.5
    for _ in range(3):
        y = y * (1.5 - half_v * y * y)
    return y


def _emb_body(x_hbm, pos_hbm, gamma_hbm, beta_hbm, tok_hbm, out_hbm,
              idx_v, pos_v, tok_v, gamma_v, beta_v, sem):
    nc = 2
    wid = lax.axis_index("s") * nc + lax.axis_index("c")
    s0w = wid * _S_PER_W

    pltpu.sync_copy(gamma_hbm, gamma_v)
    pltpu.sync_copy(beta_hbm, beta_v)

    for sc in range(_NCH_S):
        for b in range(_BATCH):
            pltpu.sync_copy(x_hbm.at[b, pl.ds(s0w + sc * _CH, _CH)],
                            idx_v.at[sc * _BATCH + b])

    def row_body(r, _):
        acc_s = jnp.zeros((_L,), jnp.float32)
        acc_q = jnp.zeros((_L,), jnp.float32)
        for j in range(_NV):
            t = tok_v[r, pl.ds(j * _L, _L)] + pos_v[r, pl.ds(j * _L, _L)]
            tok_v[r, pl.ds(j * _L, _L)] = t
            acc_s = acc_s + t
            acc_q = acc_q + t * t
        mean = _lane_sum(acc_s) * _INV_H
        ex2 = _lane_sum(acc_q) * _INV_H
        var = ex2 - mean * mean
        rstd = _rsqrt_vec(var + _EPS)
        for j in range(_NV):
            t = tok_v[r, pl.ds(j * _L, _L)]
            o = (t - mean) * rstd * gamma_v[pl.ds(j * _L, _L)] \
                + beta_v[pl.ds(j * _L, _L)]
            tok_v[r, pl.ds(j * _L, _L)] = o
        return 0

    for sc in range(_NCH_S):
        pltpu.sync_copy(pos_hbm.at[pl.ds(s0w + sc * _CH, _CH)], pos_v)
        for b in range(_BATCH):
            pltpu.async_copy(tok_hbm.at[idx_v.at[sc * _BATCH + b]],
                             tok_v, sem).wait()
            lax.fori_loop(0, _CH, row_body, 0)
            pltpu.sync_copy(tok_v,
                            out_hbm.at[b, pl.ds(s0w + sc * _CH, _CH)])


@jax.jit
def kernel(x, token_table, pos_table, gamma, beta):
    mesh = plsc.VectorSubcoreMesh(core_axis_name="c", subcore_axis_name="s")
    run = functools.partial(
        pl.kernel,
        mesh=mesh,
        out_type=jax.ShapeDtypeStruct((_BATCH, _SEQ, _HIDDEN), jnp.float32),
        scratch_types=[
            pltpu.VMEM((_NCH_S * _BATCH, _CH), jnp.int32),
            pltpu.VMEM((_CH, _HIDDEN), jnp.float32),
            pltpu.VMEM((_CH, _HIDDEN), jnp.float32),
            pltpu.VMEM((_HIDDEN,), jnp.float32),
            pltpu.VMEM((_HIDDEN,), jnp.float32),
            pltpu.SemaphoreType.DMA,
        ],
    )(_emb_body)
    return run(x, pos_table, gamma, beta, token_table)
